# baseline (speedup 1.0000x reference)
import jax
import jax.numpy as jnp
from jax.experimental import pallas as pl
from jax.experimental.pallas import tpu as pltpu

_K = 10
_B = 128
_V = 32768
_RG = 8
_LANES = 128
_NSL = _V // _LANES
_NG = _B // _RG
_NEG = -3.0e38


def _rowgroup_topk(llm_ref, slm_ref):
    init = tuple(jnp.full((_RG, _LANES), _NEG, dtype=jnp.float32)
                 for _ in range(2 * _K))

    def body(i, tiles):
        t = list(tiles)
        vl = llm_ref[:, pl.ds(i * _LANES, _LANES)]
        vs = slm_ref[:, pl.ds(i * _LANES, _LANES)]
        for k in range(_K):
            hl = jnp.maximum(t[k], vl)
            vl = jnp.minimum(t[k], vl)
            t[k] = hl
            hs = jnp.maximum(t[_K + k], vs)
            vs = jnp.minimum(t[_K + k], vs)
            t[_K + k] = hs
        return tuple(t)

    tiles = jax.lax.fori_loop(0, _NSL, body, init, unroll=8)

    def merge(ts):
        cand = jnp.concatenate(ts, axis=1)
        idx = jax.lax.broadcasted_iota(jnp.int32, cand.shape, 1)
        outs = []
        for _ in range(_K):
            m = jnp.max(cand, axis=1, keepdims=True)
            eq = cand == m
            pos = jnp.min(jnp.where(eq, idx, _K * _LANES), axis=1,
                          keepdims=True)
            cand = jnp.where(idx == pos, _NEG, cand)
            outs.append(m)
        return jnp.concatenate(outs, axis=1)

    return merge(tiles[:_K]), merge(tiles[_K:])


def _kernel_body(llm_ref, slm_ref, w1t_ref, b1_ref, w2t_ref, b2_ref,
                 w3t_ref, b3_ref, out_ref, tl_ref, ts_ref):
    g = pl.program_id(0)
    tl, ts = _rowgroup_topk(llm_ref, slm_ref)
    tl_ref[pl.ds(g * _RG, _RG), :] = tl
    ts_ref[pl.ds(g * _RG, _RG), :] = ts

    @pl.when(g == _NG - 1)
    def _():
        c = jnp.concatenate([tl_ref[...], ts_ref[...]], axis=1)
        z1 = jnp.dot(c, w1t_ref[...],
                     preferred_element_type=jnp.float32) + b1_ref[...]
        h1 = jnp.maximum(z1, 0.0)
        z2 = jnp.dot(h1, w2t_ref[...],
                     preferred_element_type=jnp.float32) + b2_ref[...]
        h2 = jnp.maximum(z2, 0.0)
        z3 = jnp.dot(h2, w3t_ref[...],
                     preferred_element_type=jnp.float32) + b3_ref[...]
        raw = jax.nn.sigmoid(z3)
        out_ref[...] = raw / jnp.sum(raw, axis=1, keepdims=True)


def kernel(llm_logits, slm_logits, W1, b1, W2, b2, W3, b3):
    llm32 = llm_logits.astype(jnp.float32)
    slm32 = slm_logits.astype(jnp.float32)
    w1t = W1.T.astype(jnp.float32)
    w2t = W2.T.astype(jnp.float32)
    w3t = W3.T.astype(jnp.float32)
    b1r = b1.reshape(1, -1).astype(jnp.float32)
    b2r = b2.reshape(1, -1).astype(jnp.float32)
    b3r = b3.reshape(1, -1).astype(jnp.float32)

    full = lambda shape: pl.BlockSpec(shape, lambda g: (0,) * len(shape))
    out = pl.pallas_call(
        _kernel_body,
        grid=(_NG,),
        in_specs=[
            pl.BlockSpec((_RG, _V), lambda g: (g, 0)),
            pl.BlockSpec((_RG, _V), lambda g: (g, 0)),
            full(w1t.shape), full(b1r.shape),
            full(w2t.shape), full(b2r.shape),
            full(w3t.shape), full(b3r.shape),
        ],
        out_specs=pl.BlockSpec((_B, 2), lambda g: (0, 0)),
        out_shape=jax.ShapeDtypeStruct((_B, 2), jnp.float32),
        scratch_shapes=[
            pltpu.VMEM((_B, _K), jnp.float32),
            pltpu.VMEM((_B, _K), jnp.float32),
        ],
    )(llm32, slm32, w1t, b1r, w2t, b2r, w3t, b3r)
    return out.astype(jnp.float16)
